# Initial kernel scaffold; baseline (speedup 1.0000x reference)
import jax
import jax.numpy as jnp
from jax.experimental import pallas as pl


def kernel(scores_map0, scores_map1, scores_pred0, scores_pred1, dispersity0, dispersity1, dist_l1, ids0_d, ids1_d, scores0, scores1, kpts01, kpts10, sim01, sim10):
    raise NotImplementedError("write your pallas kernel here")


# speedup vs baseline: 7.2387x; 7.2387x over previous
import functools

import jax
import jax.numpy as jnp
from jax import lax
from jax.experimental import pallas as pl
from jax.experimental.pallas import tpu as pltpu
from jax.experimental.pallas import tpu_sc as plsc

_TH = 0.1
_INV_T = 10.0
_PEAKY_W = 0.5
_REPROJ_W = 1.0
_SCOREMAP_W = 0.5

_B = 2
_N = 256
_H = 128
_W = 128
_D = 200
_M = 400

_NUM_TILES = 32
_PTS_PER_TILE = (_B * 2 * _N) // _NUM_TILES
_PK_PAD = 1024


def _lane_iota():
    return lax.iota(jnp.int32, 16)


def _sc_body(sim01_r, sim10_r, smap0_r, smap1_r, kx01_r, ky01_r, kx10_r,
             ky10_r, s0_r, s1_r, dist_r, ids0_r, ids1_r, pred0_r, disp0_r,
             pred1_r, disp1_r,
             part_r,
             kxa_v, kya_v, kxb_v, kyb_v, sca_v, scb_v, wx_v, wy_v,
             sidx_v, midx_v, sva_v, svb_v, mva_v, mvb_v,
             ids0_v, ids1_v, didx_v, dval_v, pa_v, pb_v,
             out_v, sem_a, sem_b, sem_c, sem_d):
    wid = lax.axis_index("c") * 16 + lax.axis_index("s")
    grp = wid // 8
    b = grp % 2
    p0 = (wid % 8) * _PTS_PER_TILE

    zf16 = jnp.zeros((16,), jnp.float32)
    for r in range(8):
        out_v[pl.ds(r * 16, 16)] = zf16

    def scoremap_part():
        df = (1 - (grp // 2)).astype(jnp.float32)
        pltpu.sync_copy(kx01_r.at[b], kxa_v)
        pltpu.sync_copy(ky01_r.at[b], kya_v)
        pltpu.sync_copy(kx10_r.at[b], kxb_v)
        pltpu.sync_copy(ky10_r.at[b], kyb_v)
        pltpu.sync_copy(s0_r.at[b], sca_v)
        pltpu.sync_copy(s1_r.at[b], scb_v)
        for c in range(2):
            kx = df * kxa_v[pl.ds(p0 + c * 16, 16)] + (1.0 - df) * kxb_v[pl.ds(p0 + c * 16, 16)]
            ky = df * kya_v[pl.ds(p0 + c * 16, 16)] + (1.0 - df) * kyb_v[pl.ds(p0 + c * 16, 16)]
            x = (kx + 1.0) * (0.5 * (_W - 1))
            y = (ky + 1.0) * (0.5 * (_H - 1))
            x0 = x.astype(jnp.int32)
            y0 = y.astype(jnp.int32)
            wx_v[pl.ds(c * 16, 16)] = x - x0.astype(jnp.float32)
            wy_v[pl.ds(c * 16, 16)] = y - y0.astype(jnp.float32)
            x0c = jnp.clip(x0, 0, _W - 1)
            x1c = jnp.clip(x0 + 1, 0, _W - 1)
            y0c = jnp.clip(y0, 0, _H - 1)
            y1c = jnp.clip(y0 + 1, 0, _H - 1)
            nvec = p0 + c * 16 + _lane_iota()
            base = (b * _N + nvec) * (_H * _W)
            row0 = base + y0c * _W
            row1 = base + y1c * _W
            sidx_v[pl.ds(c * 64 + 0, 16)] = row0 + x0c
            sidx_v[pl.ds(c * 64 + 16, 16)] = row0 + x1c
            sidx_v[pl.ds(c * 64 + 32, 16)] = row1 + x0c
            sidx_v[pl.ds(c * 64 + 48, 16)] = row1 + x1c
            mrow0 = b * (_H * _W) + y0c * _W
            mrow1 = b * (_H * _W) + y1c * _W
            midx_v[pl.ds(c * 64 + 0, 16)] = mrow0 + x0c
            midx_v[pl.ds(c * 64 + 16, 16)] = mrow0 + x1c
            midx_v[pl.ds(c * 64 + 32, 16)] = mrow1 + x0c
            midx_v[pl.ds(c * 64 + 48, 16)] = mrow1 + x1c
        cp1 = pltpu.async_copy(sim01_r.at[sidx_v], sva_v, sem_a)
        cp2 = pltpu.async_copy(sim10_r.at[sidx_v], svb_v, sem_b)
        cp3 = pltpu.async_copy(smap1_r.at[midx_v], mva_v, sem_c)
        cp4 = pltpu.async_copy(smap0_r.at[midx_v], mvb_v, sem_d)
        cp1.wait()
        cp2.wait()
        cp3.wait()
        cp4.wait()
        acc_n = jnp.zeros((16,), jnp.float32)
        acc_d = jnp.zeros((16,), jnp.float32)
        for c in range(2):
            wx1 = wx_v[pl.ds(c * 16, 16)]
            wy1 = wy_v[pl.ds(c * 16, 16)]
            wx0 = 1.0 - wx1
            wy0 = 1.0 - wy1
            w = (wy0 * wx0, wy0 * wx1, wy1 * wx0, wy1 * wx1)
            fs = jnp.zeros((16,), jnp.float32)
            sk = jnp.zeros((16,), jnp.float32)
            for k in range(4):
                sl = pl.ds(c * 64 + k * 16, 16)
                v = df * sva_v[sl] + (1.0 - df) * svb_v[sl]
                m = df * mva_v[sl] + (1.0 - df) * mvb_v[sl]
                fs = fs + w[k] * jnp.exp((v - 1.0) * _INV_T)
                sk = sk + w[k] * m
            sl16 = pl.ds(p0 + c * 16, 16)
            scv = df * sca_v[sl16] + (1.0 - df) * scb_v[sl16]
            s = sk * scv
            acc_n = acc_n + (1.0 - fs) * s
            acc_d = acc_d + s
        out_v[pl.ds(0, 16)] = acc_n
        out_v[pl.ds(16, 16)] = acc_d

    scoremap_part()

    @pl.when(wid < 2)
    def _():
        pltpu.sync_copy(ids0_r.at[wid], ids0_v)
        pltpu.sync_copy(ids1_r.at[wid], ids1_v)
        pltpu.sync_copy(s0_r.at[wid], pa_v.at[pl.ds(0, _N)])
        pltpu.sync_copy(s1_r.at[wid], pb_v.at[pl.ds(0, _N)])
        dbase = wid * (_N * _N)
        for c in range(16):
            i0 = ids0_v[pl.ds(c * 16, 16)]
            i1 = ids1_v[pl.ds(c * 16, 16)]
            didx_v[c // 8, pl.ds((c % 8) * 16, 16)] = dbase + i0 * _N + i1
        cp1 = pltpu.async_copy(dist_r.at[didx_v.at[0]], dval_v.at[0], sem_a)
        cp2 = pltpu.async_copy(dist_r.at[didx_v.at[1]], dval_v.at[1], sem_b)
        cp1.wait()
        cp2.wait()
        rs = jnp.zeros((16,), jnp.float32)
        rc = jnp.zeros((16,), jnp.float32)
        for c in range(16):
            i0 = ids0_v[pl.ds(c * 16, 16)]
            i1 = ids1_v[pl.ds(c * 16, 16)]
            s0g = plsc.load_gather(pa_v.at[pl.ds(0, _N)], [i0])
            s1g = plsc.load_gather(pb_v.at[pl.ds(0, _N)], [i1])
            inb = (c * 16 + _lane_iota()) < _D
            ok = (s0g > _TH) & (s1g > _TH) & inb
            vf = jnp.where(ok, 1.0, 0.0)
            d = dval_v[c // 8, pl.ds((c % 8) * 16, 16)]
            rs = rs + d * vf
            rc = rc + vf
        out_v[pl.ds(32, 16)] = rs
        out_v[pl.ds(48, 16)] = rc

    def peaky(pred_r, disp_r, lane0):
        pltpu.sync_copy(pred_r, pa_v)
        pltpu.sync_copy(disp_r, pb_v)
        ps = jnp.zeros((16,), jnp.float32)
        pc = jnp.zeros((16,), jnp.float32)
        for c in range(_PK_PAD // 16):
            p = pa_v[pl.ds(c * 16, 16)]
            dd = pb_v[pl.ds(c * 16, 16)]
            vf = jnp.where(p > _TH, 1.0, 0.0)
            ps = ps + dd * vf
            pc = pc + vf
        out_v[pl.ds(lane0, 16)] = ps
        out_v[pl.ds(lane0 + 16, 16)] = pc

    @pl.when(wid == 2)
    def _():
        peaky(pred0_r, disp0_r, 64)
        peaky(pred1_r, disp1_r, 96)

    pltpu.sync_copy(out_v, part_r.at[wid])


def _fin_body(part_ref, o_ref):
    p = part_ref[...]
    shp = (_NUM_TILES, 128)
    rows = lax.broadcasted_iota(jnp.int32, shp, 0)
    cols = lax.broadcasted_iota(jnp.int32, shp, 1) // 16

    def msum(mask):
        return jnp.sum(jnp.where(mask, p, 0.0))

    total = jnp.float32(0.0)
    for g in range(4):
        gm = (rows // 8) == g
        num_g = msum(gm & (cols == 0))
        den_g = msum(gm & (cols == 1))
        total = total + num_g * jnp.float32(_N) / den_g
    loss_scoremap = total / jnp.float32(_B * 2 * _N)

    rsum = msum(cols == 2)
    rcnt = msum(cols == 3)
    loss_reproj = rsum / jnp.maximum(rcnt, 1.0)

    ps0 = msum((rows == 2) & (cols == 4))
    pc0 = msum((rows == 2) & (cols == 5))
    ps1 = msum((rows == 2) & (cols == 6))
    pc1 = msum((rows == 2) & (cols == 7))
    loss_peaky = (ps0 / jnp.maximum(pc0, 1.0) + ps1 / jnp.maximum(pc1, 1.0)) / 2.0

    out = (_PEAKY_W * loss_peaky + _REPROJ_W * loss_reproj +
           _SCOREMAP_W * loss_scoremap)
    o_ref[...] = jnp.reshape(out, (1, 1))


@jax.jit
def _detector_loss(scores_map0, scores_map1, scores_pred0, scores_pred1,
                   dispersity0, dispersity1, dist_l1, ids0_d, ids1_d,
                   scores0, scores1, kpts01, kpts10, sim01, sim10):
    sim01_f = sim01.reshape(-1)
    sim10_f = sim10.reshape(-1)
    smap0_f = scores_map0.reshape(-1)
    smap1_f = scores_map1.reshape(-1)
    dist_f = dist_l1.reshape(-1)
    kx01 = kpts01[..., 0]
    ky01 = kpts01[..., 1]
    kx10 = kpts10[..., 0]
    ky10 = kpts10[..., 1]
    pk_pad = _PK_PAD - _B * _M
    pred0_f = jnp.pad(scores_pred0.reshape(-1), (0, pk_pad))
    disp0_f = jnp.pad(dispersity0.reshape(-1), (0, pk_pad))
    pred1_f = jnp.pad(scores_pred1.reshape(-1), (0, pk_pad))
    disp1_f = jnp.pad(dispersity1.reshape(-1), (0, pk_pad))
    ids0 = jnp.pad(ids0_d.astype(jnp.int32), ((0, 0), (0, _N - _D)))
    ids1 = jnp.pad(ids1_d.astype(jnp.int32), ((0, 0), (0, _N - _D)))

    mesh = plsc.VectorSubcoreMesh(core_axis_name="c", subcore_axis_name="s")
    sc_fn = pl.kernel(
        _sc_body,
        out_type=jax.ShapeDtypeStruct((_NUM_TILES, 128), jnp.float32),
        mesh=mesh,
        compiler_params=pltpu.CompilerParams(needs_layout_passes=False),
        scratch_types=[
            pltpu.VMEM((_N,), jnp.float32),
            pltpu.VMEM((_N,), jnp.float32),
            pltpu.VMEM((_N,), jnp.float32),
            pltpu.VMEM((_N,), jnp.float32),
            pltpu.VMEM((_N,), jnp.float32),
            pltpu.VMEM((_N,), jnp.float32),
            pltpu.VMEM((_PTS_PER_TILE,), jnp.float32),
            pltpu.VMEM((_PTS_PER_TILE,), jnp.float32),
            pltpu.VMEM((4 * _PTS_PER_TILE,), jnp.int32),
            pltpu.VMEM((4 * _PTS_PER_TILE,), jnp.int32),
            pltpu.VMEM((4 * _PTS_PER_TILE,), jnp.float32),
            pltpu.VMEM((4 * _PTS_PER_TILE,), jnp.float32),
            pltpu.VMEM((4 * _PTS_PER_TILE,), jnp.float32),
            pltpu.VMEM((4 * _PTS_PER_TILE,), jnp.float32),
            pltpu.VMEM((_N,), jnp.int32),
            pltpu.VMEM((_N,), jnp.int32),
            pltpu.VMEM((2, 128), jnp.int32),
            pltpu.VMEM((2, 128), jnp.float32),
            pltpu.VMEM((_PK_PAD,), jnp.float32),
            pltpu.VMEM((_PK_PAD,), jnp.float32),
            pltpu.VMEM((128,), jnp.float32),
            pltpu.SemaphoreType.DMA,
            pltpu.SemaphoreType.DMA,
            pltpu.SemaphoreType.DMA,
            pltpu.SemaphoreType.DMA,
        ],
    )
    partials = sc_fn(sim01_f, sim10_f, smap0_f, smap1_f, kx01, ky01, kx10,
                     ky10, scores0, scores1, dist_f, ids0, ids1, pred0_f,
                     disp0_f, pred1_f, disp1_f)

    loss = pl.pallas_call(
        _fin_body,
        out_shape=jax.ShapeDtypeStruct((1, 1), jnp.float32),
    )(partials)
    return loss[0, 0]


def kernel(scores_map0, scores_map1, scores_pred0, scores_pred1, dispersity0,
           dispersity1, dist_l1, ids0_d, ids1_d, scores0, scores1, kpts01,
           kpts10, sim01, sim10):
    assert sim01.shape == (_B, _N, _H, _W)
    assert ids0_d.shape == (_B, _D)
    assert scores_pred0.shape == (_B, _M)
    return _detector_loss(scores_map0, scores_map1, scores_pred0,
                          scores_pred1, dispersity0, dispersity1, dist_l1,
                          ids0_d, ids1_d, scores0, scores1, kpts01, kpts10,
                          sim01, sim10)


# speedup vs baseline: 9.3790x; 1.2957x over previous
import functools

import jax
import jax.numpy as jnp
from jax import lax
from jax.experimental import pallas as pl
from jax.experimental.pallas import tpu as pltpu
from jax.experimental.pallas import tpu_sc as plsc

_TH = 0.1
_INV_T = 10.0
_PEAKY_W = 0.5
_REPROJ_W = 1.0
_SCOREMAP_W = 0.5

_B = 2
_N = 256
_H = 128
_W = 128
_D = 200
_M = 400

_NUM_TILES = 32
_PTS_PER_TILE = (_B * 2 * _N) // _NUM_TILES
_PK_PAD = 1024


def _lane_iota():
    return lax.iota(jnp.int32, 16)


def _sc_body(sim01_r, sim10_r, smap0_r, smap1_r, kp01_r, kp10_r,
             s0_r, s1_r, dist_r, ids0_r, ids1_r,
             part_r,
             kpa_v, kpb_v, sca_v, scb_v, wx_v, wy_v,
             sidx_v, midx_v, sva_v, svb_v, mva_v, mvb_v,
             ids0_v, ids1_v, didx_v, dval_v,
             out_v, sem_a, sem_b, sem_c, sem_d):
    wid = lax.axis_index("c") * 16 + lax.axis_index("s")
    grp = wid // 8
    b = grp % 2
    p0 = (wid % 8) * _PTS_PER_TILE

    zf16 = jnp.zeros((16,), jnp.float32)
    for r in range(8):
        out_v[pl.ds(r * 16, 16)] = zf16

    df = (1 - (grp // 2)).astype(jnp.float32)
    cp1 = pltpu.async_copy(kp01_r.at[b], kpa_v, sem_a)
    cp2 = pltpu.async_copy(kp10_r.at[b], kpb_v, sem_b)
    cp3 = pltpu.async_copy(s0_r.at[b], sca_v, sem_c)
    cp4 = pltpu.async_copy(s1_r.at[b], scb_v, sem_d)
    cp1.wait()
    cp2.wait()
    cp3.wait()
    cp4.wait()
    for c in range(2):
        nvec = p0 + c * 16 + _lane_iota()
        nv2 = nvec * 2
        kx = df * plsc.load_gather(kpa_v, [nv2]) + \
            (1.0 - df) * plsc.load_gather(kpb_v, [nv2])
        ky = df * plsc.load_gather(kpa_v, [nv2 + 1]) + \
            (1.0 - df) * plsc.load_gather(kpb_v, [nv2 + 1])
        x = (kx + 1.0) * (0.5 * (_W - 1))
        y = (ky + 1.0) * (0.5 * (_H - 1))
        x0 = x.astype(jnp.int32)
        y0 = y.astype(jnp.int32)
        wx_v[pl.ds(c * 16, 16)] = x - x0.astype(jnp.float32)
        wy_v[pl.ds(c * 16, 16)] = y - y0.astype(jnp.float32)
        x0c = jnp.clip(x0, 0, _W - 1)
        x1c = jnp.clip(x0 + 1, 0, _W - 1)
        y0c = jnp.clip(y0, 0, _H - 1)
        y1c = jnp.clip(y0 + 1, 0, _H - 1)
        base = (b * _N + nvec) * (_H * _W)
        row0 = base + y0c * _W
        row1 = base + y1c * _W
        sidx_v[pl.ds(c * 64 + 0, 16)] = row0 + x0c
        sidx_v[pl.ds(c * 64 + 16, 16)] = row0 + x1c
        sidx_v[pl.ds(c * 64 + 32, 16)] = row1 + x0c
        sidx_v[pl.ds(c * 64 + 48, 16)] = row1 + x1c
        mrow0 = b * (_H * _W) + y0c * _W
        mrow1 = b * (_H * _W) + y1c * _W
        midx_v[pl.ds(c * 64 + 0, 16)] = mrow0 + x0c
        midx_v[pl.ds(c * 64 + 16, 16)] = mrow0 + x1c
        midx_v[pl.ds(c * 64 + 32, 16)] = mrow1 + x0c
        midx_v[pl.ds(c * 64 + 48, 16)] = mrow1 + x1c
    cp1 = pltpu.async_copy(sim01_r.at[sidx_v], sva_v, sem_a)
    cp2 = pltpu.async_copy(sim10_r.at[sidx_v], svb_v, sem_b)
    cp3 = pltpu.async_copy(smap1_r.at[midx_v], mva_v, sem_c)
    cp4 = pltpu.async_copy(smap0_r.at[midx_v], mvb_v, sem_d)
    cp1.wait()
    cp2.wait()
    cp3.wait()
    cp4.wait()
    acc_n = jnp.zeros((16,), jnp.float32)
    acc_d = jnp.zeros((16,), jnp.float32)
    for c in range(2):
        wx1 = wx_v[pl.ds(c * 16, 16)]
        wy1 = wy_v[pl.ds(c * 16, 16)]
        wx0 = 1.0 - wx1
        wy0 = 1.0 - wy1
        w = (wy0 * wx0, wy0 * wx1, wy1 * wx0, wy1 * wx1)
        fs = jnp.zeros((16,), jnp.float32)
        sk = jnp.zeros((16,), jnp.float32)
        for k in range(4):
            sl = pl.ds(c * 64 + k * 16, 16)
            v = df * sva_v[sl] + (1.0 - df) * svb_v[sl]
            m = df * mva_v[sl] + (1.0 - df) * mvb_v[sl]
            fs = fs + w[k] * jnp.exp((v - 1.0) * _INV_T)
            sk = sk + w[k] * m
        nv2 = (p0 + c * 16 + _lane_iota())
        scv = df * plsc.load_gather(sca_v, [nv2]) + \
            (1.0 - df) * plsc.load_gather(scb_v, [nv2])
        s = sk * scv
        acc_n = acc_n + (1.0 - fs) * s
        acc_d = acc_d + s
    out_v[pl.ds(0, 16)] = acc_n
    out_v[pl.ds(16, 16)] = acc_d

    @pl.when((wid == 0) | (wid == 24))
    def _():
        b_r = wid // 24
        cp1 = pltpu.async_copy(ids0_r.at[b_r], ids0_v, sem_a)
        cp2 = pltpu.async_copy(ids1_r.at[b_r], ids1_v, sem_b)
        cp1.wait()
        cp2.wait()
        dbase = b_r * (_N * _N)
        for c in range(16):
            i0 = ids0_v[pl.ds(c * 16, 16)]
            i1 = ids1_v[pl.ds(c * 16, 16)]
            didx_v[c // 8, pl.ds((c % 8) * 16, 16)] = dbase + i0 * _N + i1
        cp1 = pltpu.async_copy(dist_r.at[didx_v.at[0]], dval_v.at[0], sem_a)
        cp2 = pltpu.async_copy(dist_r.at[didx_v.at[1]], dval_v.at[1], sem_b)
        cp1.wait()
        cp2.wait()
        rs = jnp.zeros((16,), jnp.float32)
        rc = jnp.zeros((16,), jnp.float32)
        for c in range(16):
            i0 = ids0_v[pl.ds(c * 16, 16)]
            i1 = ids1_v[pl.ds(c * 16, 16)]
            s0g = plsc.load_gather(sca_v, [i0])
            s1g = plsc.load_gather(scb_v, [i1])
            inb = (c * 16 + _lane_iota()) < _D
            ok = (s0g > _TH) & (s1g > _TH) & inb
            vf = jnp.where(ok, 1.0, 0.0)
            d = dval_v[c // 8, pl.ds((c % 8) * 16, 16)]
            rs = rs + d * vf
            rc = rc + vf
        out_v[pl.ds(32, 16)] = rs
        out_v[pl.ds(48, 16)] = rc

    pltpu.sync_copy(out_v, part_r.at[wid])


def _fin_body(part_ref, pred0_ref, disp0_ref, pred1_ref, disp1_ref, o_ref):
    p = part_ref[...]
    shp = (_NUM_TILES, 128)
    rows = lax.broadcasted_iota(jnp.int32, shp, 0)
    cols = lax.broadcasted_iota(jnp.int32, shp, 1) // 16

    def msum(mask):
        return jnp.sum(jnp.where(mask, p, 0.0))

    total = jnp.float32(0.0)
    for g in range(4):
        gm = (rows // 8) == g
        num_g = msum(gm & (cols == 0))
        den_g = msum(gm & (cols == 1))
        total = total + num_g * jnp.float32(_N) / den_g
    loss_scoremap = total / jnp.float32(_B * 2 * _N)

    rsum = msum(cols == 2)
    rcnt = msum(cols == 3)
    loss_reproj = rsum / jnp.maximum(rcnt, 1.0)

    def pk(pred, disp):
        vf = jnp.where(pred > _TH, 1.0, 0.0)
        return jnp.sum(disp * vf) / jnp.maximum(jnp.sum(vf), 1.0)

    loss_peaky = (pk(pred0_ref[...], disp0_ref[...]) +
                  pk(pred1_ref[...], disp1_ref[...])) / 2.0

    out = (_PEAKY_W * loss_peaky + _REPROJ_W * loss_reproj +
           _SCOREMAP_W * loss_scoremap)
    o_ref[...] = jnp.reshape(out, (1, 1))


@jax.jit
def _detector_loss(scores_map0, scores_map1, scores_pred0, scores_pred1,
                   dispersity0, dispersity1, dist_l1, ids0_d, ids1_d,
                   scores0, scores1, kpts01, kpts10, sim01, sim10):
    sim01_f = sim01.reshape(-1)
    sim10_f = sim10.reshape(-1)
    smap0_f = scores_map0.reshape(-1)
    smap1_f = scores_map1.reshape(-1)
    dist_f = dist_l1.reshape(-1)
    kp01 = kpts01.reshape(_B, 2 * _N)
    kp10 = kpts10.reshape(_B, 2 * _N)
    ids0 = jnp.pad(ids0_d.astype(jnp.int32), ((0, 0), (0, _N - _D)))
    ids1 = jnp.pad(ids1_d.astype(jnp.int32), ((0, 0), (0, _N - _D)))

    mesh = plsc.VectorSubcoreMesh(core_axis_name="c", subcore_axis_name="s")
    sc_fn = pl.kernel(
        _sc_body,
        out_type=jax.ShapeDtypeStruct((_NUM_TILES, 128), jnp.float32),
        mesh=mesh,
        compiler_params=pltpu.CompilerParams(needs_layout_passes=False),
        scratch_types=[
            pltpu.VMEM((2 * _N,), jnp.float32),
            pltpu.VMEM((2 * _N,), jnp.float32),
            pltpu.VMEM((_N,), jnp.float32),
            pltpu.VMEM((_N,), jnp.float32),
            pltpu.VMEM((_PTS_PER_TILE,), jnp.float32),
            pltpu.VMEM((_PTS_PER_TILE,), jnp.float32),
            pltpu.VMEM((4 * _PTS_PER_TILE,), jnp.int32),
            pltpu.VMEM((4 * _PTS_PER_TILE,), jnp.int32),
            pltpu.VMEM((4 * _PTS_PER_TILE,), jnp.float32),
            pltpu.VMEM((4 * _PTS_PER_TILE,), jnp.float32),
            pltpu.VMEM((4 * _PTS_PER_TILE,), jnp.float32),
            pltpu.VMEM((4 * _PTS_PER_TILE,), jnp.float32),
            pltpu.VMEM((_N,), jnp.int32),
            pltpu.VMEM((_N,), jnp.int32),
            pltpu.VMEM((2, 128), jnp.int32),
            pltpu.VMEM((2, 128), jnp.float32),
            pltpu.VMEM((128,), jnp.float32),
            pltpu.SemaphoreType.DMA,
            pltpu.SemaphoreType.DMA,
            pltpu.SemaphoreType.DMA,
            pltpu.SemaphoreType.DMA,
        ],
    )
    partials = sc_fn(sim01_f, sim10_f, smap0_f, smap1_f, kp01, kp10,
                     scores0, scores1, dist_f, ids0, ids1)

    loss = pl.pallas_call(
        _fin_body,
        out_shape=jax.ShapeDtypeStruct((1, 1), jnp.float32),
    )(partials, scores_pred0, dispersity0, scores_pred1, dispersity1)
    return loss[0, 0]


def kernel(scores_map0, scores_map1, scores_pred0, scores_pred1, dispersity0,
           dispersity1, dist_l1, ids0_d, ids1_d, scores0, scores1, kpts01,
           kpts10, sim01, sim10):
    assert sim01.shape == (_B, _N, _H, _W)
    assert ids0_d.shape == (_B, _D)
    assert scores_pred0.shape == (_B, _M)
    return _detector_loss(scores_map0, scores_map1, scores_pred0,
                          scores_pred1, dispersity0, dispersity1, dist_l1,
                          ids0_d, ids1_d, scores0, scores1, kpts01, kpts10,
                          sim01, sim10)


# speedup vs baseline: 9.4275x; 1.0052x over previous
import functools

import jax
import jax.numpy as jnp
from jax import lax
from jax.experimental import pallas as pl
from jax.experimental.pallas import tpu as pltpu
from jax.experimental.pallas import tpu_sc as plsc

_TH = 0.1
_INV_T = 10.0
_PEAKY_W = 0.5
_REPROJ_W = 1.0
_SCOREMAP_W = 0.5

_B = 2
_N = 256
_H = 128
_W = 128
_D = 200
_M = 400

_NUM_TILES = 32
_PTS_PER_TILE = (_B * 2 * _N) // _NUM_TILES
_PK_PAD = 1024


def _lane_iota():
    return lax.iota(jnp.int32, 16)


def _sc_body(sim01_r, sim10_r, smap0_r, smap1_r, kp01_r, kp10_r,
             s0_r, s1_r, dist_r, ids0_r, ids1_r,
             part_r,
             kpa_v, kpb_v, sca_v, scb_v, wx_v, wy_v,
             sidx_v, midx_v, sva_v, svb_v, mva_v, mvb_v,
             ids0_v, ids1_v, didx_v, dval_v,
             out_v, sem_a, sem_b, sem_c, sem_d):
    wid = lax.axis_index("c") * 16 + lax.axis_index("s")
    grp = wid // 8
    b = grp % 2
    p0 = (wid % 8) * _PTS_PER_TILE

    zf16 = jnp.zeros((16,), jnp.float32)
    for r in range(8):
        out_v[pl.ds(r * 16, 16)] = zf16

    df = (1 - (grp // 2)).astype(jnp.float32)
    cp1 = pltpu.async_copy(kp01_r.at[b], kpa_v, sem_a)
    cp2 = pltpu.async_copy(kp10_r.at[b], kpb_v, sem_b)
    cp3 = pltpu.async_copy(s0_r.at[b], sca_v, sem_c)
    cp4 = pltpu.async_copy(s1_r.at[b], scb_v, sem_d)
    cp1.wait()
    cp2.wait()
    cp3.wait()
    cp4.wait()
    for c in range(2):
        nvec = p0 + c * 16 + _lane_iota()
        nv2 = nvec * 2
        kx = df * plsc.load_gather(kpa_v, [nv2]) + \
            (1.0 - df) * plsc.load_gather(kpb_v, [nv2])
        ky = df * plsc.load_gather(kpa_v, [nv2 + 1]) + \
            (1.0 - df) * plsc.load_gather(kpb_v, [nv2 + 1])
        x = (kx + 1.0) * (0.5 * (_W - 1))
        y = (ky + 1.0) * (0.5 * (_H - 1))
        x0 = x.astype(jnp.int32)
        y0 = y.astype(jnp.int32)
        wx_v[pl.ds(c * 16, 16)] = x - x0.astype(jnp.float32)
        wy_v[pl.ds(c * 16, 16)] = y - y0.astype(jnp.float32)
        x0c = jnp.clip(x0, 0, _W - 1)
        x1c = jnp.clip(x0 + 1, 0, _W - 1)
        y0c = jnp.clip(y0, 0, _H - 1)
        y1c = jnp.clip(y0 + 1, 0, _H - 1)
        base = (b * _N + nvec) * (_H * _W)
        row0 = base + y0c * _W
        row1 = base + y1c * _W
        sidx_v[pl.ds(c * 64 + 0, 16)] = row0 + x0c
        sidx_v[pl.ds(c * 64 + 16, 16)] = row0 + x1c
        sidx_v[pl.ds(c * 64 + 32, 16)] = row1 + x0c
        sidx_v[pl.ds(c * 64 + 48, 16)] = row1 + x1c
        mrow0 = b * (_H * _W) + y0c * _W
        mrow1 = b * (_H * _W) + y1c * _W
        midx_v[pl.ds(c * 64 + 0, 16)] = mrow0 + x0c
        midx_v[pl.ds(c * 64 + 16, 16)] = mrow0 + x1c
        midx_v[pl.ds(c * 64 + 32, 16)] = mrow1 + x0c
        midx_v[pl.ds(c * 64 + 48, 16)] = mrow1 + x1c
    cp1 = pltpu.async_copy(sim01_r.at[sidx_v], sva_v, sem_a)
    cp2 = pltpu.async_copy(sim10_r.at[sidx_v], svb_v, sem_b)
    cp3 = pltpu.async_copy(smap1_r.at[midx_v], mva_v, sem_c)
    cp4 = pltpu.async_copy(smap0_r.at[midx_v], mvb_v, sem_d)
    cp1.wait()
    cp2.wait()
    cp3.wait()
    cp4.wait()
    acc_n = jnp.zeros((16,), jnp.float32)
    acc_d = jnp.zeros((16,), jnp.float32)
    for c in range(2):
        wx1 = wx_v[pl.ds(c * 16, 16)]
        wy1 = wy_v[pl.ds(c * 16, 16)]
        wx0 = 1.0 - wx1
        wy0 = 1.0 - wy1
        w = (wy0 * wx0, wy0 * wx1, wy1 * wx0, wy1 * wx1)
        fs = jnp.zeros((16,), jnp.float32)
        sk = jnp.zeros((16,), jnp.float32)
        for k in range(4):
            sl = pl.ds(c * 64 + k * 16, 16)
            v = df * sva_v[sl] + (1.0 - df) * svb_v[sl]
            m = df * mva_v[sl] + (1.0 - df) * mvb_v[sl]
            fs = fs + w[k] * jnp.exp((v - 1.0) * _INV_T)
            sk = sk + w[k] * m
        nv2 = (p0 + c * 16 + _lane_iota())
        scv = df * plsc.load_gather(sca_v, [nv2]) + \
            (1.0 - df) * plsc.load_gather(scb_v, [nv2])
        s = sk * scv
        acc_n = acc_n + (1.0 - fs) * s
        acc_d = acc_d + s
    out_v[pl.ds(0, 16)] = acc_n
    out_v[pl.ds(16, 16)] = acc_d

    @pl.when((wid == 0) | (wid == 24))
    def _():
        b_r = wid // 24
        cp1 = pltpu.async_copy(ids0_r, ids0_v, sem_a)
        cp2 = pltpu.async_copy(ids1_r, ids1_v, sem_b)
        cp1.wait()
        cp2.wait()
        dbase = b_r * (_N * _N)
        i_off = b_r * _D
        nchunks = -(-_D // 16)
        for c in range(nchunks):
            gidx = jnp.minimum(i_off + c * 16 + _lane_iota(),
                               _B * _D - 1)
            i0 = plsc.load_gather(ids0_v, [gidx])
            i1 = plsc.load_gather(ids1_v, [gidx])
            didx_v[c // 8, pl.ds((c % 8) * 16, 16)] = dbase + i0 * _N + i1
        cp1 = pltpu.async_copy(dist_r.at[didx_v.at[0]], dval_v.at[0], sem_a)
        cp2 = pltpu.async_copy(dist_r.at[didx_v.at[1]], dval_v.at[1], sem_b)
        cp1.wait()
        cp2.wait()
        rs = jnp.zeros((16,), jnp.float32)
        rc = jnp.zeros((16,), jnp.float32)
        for c in range(nchunks):
            gidx = jnp.minimum(i_off + c * 16 + _lane_iota(),
                               _B * _D - 1)
            i0 = plsc.load_gather(ids0_v, [gidx])
            i1 = plsc.load_gather(ids1_v, [gidx])
            s0g = plsc.load_gather(sca_v, [i0])
            s1g = plsc.load_gather(scb_v, [i1])
            inb = (c * 16 + _lane_iota()) < _D
            ok = (s0g > _TH) & (s1g > _TH) & inb
            vf = jnp.where(ok, 1.0, 0.0)
            d = dval_v[c // 8, pl.ds((c % 8) * 16, 16)]
            rs = rs + d * vf
            rc = rc + vf
        out_v[pl.ds(32, 16)] = rs
        out_v[pl.ds(48, 16)] = rc

    pltpu.sync_copy(out_v, part_r.at[wid])


def _fin_body(part_ref, pred0_ref, disp0_ref, pred1_ref, disp1_ref, o_ref):
    p = part_ref[...]
    shp = (_NUM_TILES, 128)
    rows = lax.broadcasted_iota(jnp.int32, shp, 0)
    cols = lax.broadcasted_iota(jnp.int32, shp, 1) // 16

    def msum(mask):
        return jnp.sum(jnp.where(mask, p, 0.0))

    total = jnp.float32(0.0)
    for g in range(4):
        gm = (rows // 8) == g
        num_g = msum(gm & (cols == 0))
        den_g = msum(gm & (cols == 1))
        total = total + num_g * jnp.float32(_N) / den_g
    loss_scoremap = total / jnp.float32(_B * 2 * _N)

    rsum = msum(cols == 2)
    rcnt = msum(cols == 3)
    loss_reproj = rsum / jnp.maximum(rcnt, 1.0)

    def pk(pred, disp):
        vf = jnp.where(pred > _TH, 1.0, 0.0)
        return jnp.sum(disp * vf) / jnp.maximum(jnp.sum(vf), 1.0)

    loss_peaky = (pk(pred0_ref[...], disp0_ref[...]) +
                  pk(pred1_ref[...], disp1_ref[...])) / 2.0

    o_ref[0] = (_PEAKY_W * loss_peaky + _REPROJ_W * loss_reproj +
                _SCOREMAP_W * loss_scoremap)


@jax.jit
def _detector_loss(scores_map0, scores_map1, scores_pred0, scores_pred1,
                   dispersity0, dispersity1, dist_l1, ids0_d, ids1_d,
                   scores0, scores1, kpts01, kpts10, sim01, sim10):
    sim01_f = sim01.reshape(-1)
    sim10_f = sim10.reshape(-1)
    smap0_f = scores_map0.reshape(-1)
    smap1_f = scores_map1.reshape(-1)
    dist_f = dist_l1.reshape(-1)
    kp01 = kpts01.reshape(_B, 2 * _N)
    kp10 = kpts10.reshape(_B, 2 * _N)
    ids0 = ids0_d.astype(jnp.int32).reshape(-1)
    ids1 = ids1_d.astype(jnp.int32).reshape(-1)

    mesh = plsc.VectorSubcoreMesh(core_axis_name="c", subcore_axis_name="s")
    sc_fn = pl.kernel(
        _sc_body,
        out_type=jax.ShapeDtypeStruct((_NUM_TILES, 128), jnp.float32),
        mesh=mesh,
        compiler_params=pltpu.CompilerParams(needs_layout_passes=False),
        scratch_types=[
            pltpu.VMEM((2 * _N,), jnp.float32),
            pltpu.VMEM((2 * _N,), jnp.float32),
            pltpu.VMEM((_N,), jnp.float32),
            pltpu.VMEM((_N,), jnp.float32),
            pltpu.VMEM((_PTS_PER_TILE,), jnp.float32),
            pltpu.VMEM((_PTS_PER_TILE,), jnp.float32),
            pltpu.VMEM((4 * _PTS_PER_TILE,), jnp.int32),
            pltpu.VMEM((4 * _PTS_PER_TILE,), jnp.int32),
            pltpu.VMEM((4 * _PTS_PER_TILE,), jnp.float32),
            pltpu.VMEM((4 * _PTS_PER_TILE,), jnp.float32),
            pltpu.VMEM((4 * _PTS_PER_TILE,), jnp.float32),
            pltpu.VMEM((4 * _PTS_PER_TILE,), jnp.float32),
            pltpu.VMEM((_B * _D,), jnp.int32),
            pltpu.VMEM((_B * _D,), jnp.int32),
            pltpu.VMEM((2, 128), jnp.int32),
            pltpu.VMEM((2, 128), jnp.float32),
            pltpu.VMEM((128,), jnp.float32),
            pltpu.SemaphoreType.DMA,
            pltpu.SemaphoreType.DMA,
            pltpu.SemaphoreType.DMA,
            pltpu.SemaphoreType.DMA,
        ],
    )
    partials = sc_fn(sim01_f, sim10_f, smap0_f, smap1_f, kp01, kp10,
                     scores0, scores1, dist_f, ids0, ids1)

    loss = pl.pallas_call(
        _fin_body,
        out_shape=jax.ShapeDtypeStruct((1,), jnp.float32),
        out_specs=pl.BlockSpec(memory_space=pltpu.SMEM),
    )(partials, scores_pred0, dispersity0, scores_pred1, dispersity1)
    return loss[0]


def kernel(scores_map0, scores_map1, scores_pred0, scores_pred1, dispersity0,
           dispersity1, dist_l1, ids0_d, ids1_d, scores0, scores1, kpts01,
           kpts10, sim01, sim10):
    assert sim01.shape == (_B, _N, _H, _W)
    assert ids0_d.shape == (_B, _D)
    assert scores_pred0.shape == (_B, _M)
    return _detector_loss(scores_map0, scores_map1, scores_pred0,
                          scores_pred1, dispersity0, dispersity1, dist_l1,
                          ids0_d, ids1_d, scores0, scores1, kpts01, kpts10,
                          sim01, sim10)


# speedup vs baseline: 10.3338x; 1.0961x over previous
import functools

import jax
import jax.numpy as jnp
from jax import lax
from jax.experimental import pallas as pl
from jax.experimental.pallas import tpu as pltpu
from jax.experimental.pallas import tpu_sc as plsc

_TH = 0.1
_INV_T = 10.0
_PEAKY_W = 0.5
_REPROJ_W = 1.0
_SCOREMAP_W = 0.5

_B = 2
_N = 256
_H = 128
_W = 128
_D = 200
_M = 400

_NUM_TILES = 32
_PTS_PER_TILE = (_B * 2 * _N) // _NUM_TILES
_PK_PAD = 1024


def _lane_iota():
    return lax.iota(jnp.int32, 16)


def _sc_body(sim01_r, sim10_r, smap0_r, smap1_r, aux_r, ids_r, dist_r,
             part_r,
             kx_v, ky_v, sc_v, wx_v, wy_v,
             sidx_v, midx_v, sva_v, svb_v, mva_v, mvb_v,
             ids0_v, ids1_v, ra_v, rb_v, dist_v,
             out_v, sem_a, sem_b, sem_c, sem_d, sem_e, sem_f, sem_g):
    wid = lax.axis_index("c") * 16 + lax.axis_index("s")
    grp = wid // 8
    b = grp % 2
    dirv = grp // 2
    p0 = (wid % 8) * _PTS_PER_TILE
    is_reproj = (wid == 0) | (wid == 24)
    b_r = wid // 24

    zf16 = jnp.zeros((16,), jnp.float32)
    for r in range(8):
        out_v[pl.ds(r * 16, 16)] = zf16

    @pl.when(is_reproj)
    def _():
        pltpu.async_copy(dist_r.at[pl.ds(b_r * _N, _N)], dist_v, sem_e)
        pltpu.async_copy(ids_r.at[b_r], ids0_v, sem_f)
        pltpu.async_copy(ids_r.at[2 + b_r], ids1_v, sem_g)

    kxrow = 4 * dirv + b
    scrow = 8 + 2 * dirv + b
    cp1 = pltpu.async_copy(aux_r.at[kxrow], kx_v, sem_a)
    cp2 = pltpu.async_copy(aux_r.at[kxrow + 2], ky_v, sem_b)
    cp3 = pltpu.async_copy(aux_r.at[scrow], sc_v, sem_c)
    cp1.wait()
    cp2.wait()
    cp3.wait()
    df = (1 - dirv).astype(jnp.float32)
    for c in range(2):
        nvec = p0 + c * 16 + _lane_iota()
        kx = kx_v[pl.ds(p0 + c * 16, 16)]
        ky = ky_v[pl.ds(p0 + c * 16, 16)]
        x = (kx + 1.0) * (0.5 * (_W - 1))
        y = (ky + 1.0) * (0.5 * (_H - 1))
        x0 = x.astype(jnp.int32)
        y0 = y.astype(jnp.int32)
        wx_v[pl.ds(c * 16, 16)] = x - x0.astype(jnp.float32)
        wy_v[pl.ds(c * 16, 16)] = y - y0.astype(jnp.float32)
        x0c = jnp.clip(x0, 0, _W - 1)
        x1c = jnp.clip(x0 + 1, 0, _W - 1)
        y0c = jnp.clip(y0, 0, _H - 1)
        y1c = jnp.clip(y0 + 1, 0, _H - 1)
        base = (b * _N + nvec) * (_H * _W)
        row0 = base + y0c * _W
        row1 = base + y1c * _W
        sidx_v[pl.ds(c * 64 + 0, 16)] = row0 + x0c
        sidx_v[pl.ds(c * 64 + 16, 16)] = row0 + x1c
        sidx_v[pl.ds(c * 64 + 32, 16)] = row1 + x0c
        sidx_v[pl.ds(c * 64 + 48, 16)] = row1 + x1c
        mrow0 = b * (_H * _W) + y0c * _W
        mrow1 = b * (_H * _W) + y1c * _W
        midx_v[pl.ds(c * 64 + 0, 16)] = mrow0 + x0c
        midx_v[pl.ds(c * 64 + 16, 16)] = mrow0 + x1c
        midx_v[pl.ds(c * 64 + 32, 16)] = mrow1 + x0c
        midx_v[pl.ds(c * 64 + 48, 16)] = mrow1 + x1c
    cp1 = pltpu.async_copy(sim01_r.at[sidx_v], sva_v, sem_a)
    cp2 = pltpu.async_copy(sim10_r.at[sidx_v], svb_v, sem_b)
    cp3 = pltpu.async_copy(smap1_r.at[midx_v], mva_v, sem_c)
    cp4 = pltpu.async_copy(smap0_r.at[midx_v], mvb_v, sem_d)
    cp1.wait()
    cp2.wait()
    cp3.wait()
    cp4.wait()
    acc_n = jnp.zeros((16,), jnp.float32)
    acc_d = jnp.zeros((16,), jnp.float32)
    for c in range(2):
        wx1 = wx_v[pl.ds(c * 16, 16)]
        wy1 = wy_v[pl.ds(c * 16, 16)]
        wx0 = 1.0 - wx1
        wy0 = 1.0 - wy1
        w = (wy0 * wx0, wy0 * wx1, wy1 * wx0, wy1 * wx1)
        fs = jnp.zeros((16,), jnp.float32)
        sk = jnp.zeros((16,), jnp.float32)
        for k in range(4):
            sl = pl.ds(c * 64 + k * 16, 16)
            v = df * sva_v[sl] + (1.0 - df) * svb_v[sl]
            m = df * mva_v[sl] + (1.0 - df) * mvb_v[sl]
            fs = fs + w[k] * jnp.exp((v - 1.0) * _INV_T)
            sk = sk + w[k] * m
        s = sk * sc_v[pl.ds(p0 + c * 16, 16)]
        acc_n = acc_n + (1.0 - fs) * s
        acc_d = acc_d + s
    out_v[pl.ds(0, 16)] = acc_n
    out_v[pl.ds(16, 16)] = acc_d

    @pl.when(is_reproj)
    def _():
        cp1 = pltpu.async_copy(aux_r.at[8 + b_r], ra_v, sem_a)
        cp2 = pltpu.async_copy(aux_r.at[10 + b_r], rb_v, sem_b)
        cp1.wait()
        cp2.wait()
        pltpu.make_async_copy(dist_r.at[pl.ds(b_r * _N, _N)], dist_v, sem_e).wait()
        pltpu.make_async_copy(ids_r.at[b_r], ids0_v, sem_f).wait()
        pltpu.make_async_copy(ids_r.at[2 + b_r], ids1_v, sem_g).wait()
        rs = jnp.zeros((16,), jnp.float32)
        rc = jnp.zeros((16,), jnp.float32)
        for c in range(-(-_D // 16)):
            gidx = c * 16 + _lane_iota()
            i0 = plsc.load_gather(ids0_v, [gidx])
            i1 = plsc.load_gather(ids1_v, [gidx])
            d = plsc.load_gather(dist_v, [i0, i1])
            s0g = plsc.load_gather(ra_v, [i0])
            s1g = plsc.load_gather(rb_v, [i1])
            ok = (s0g > _TH) & (s1g > _TH) & (gidx < _D)
            vf = jnp.where(ok, 1.0, 0.0)
            rs = rs + d * vf
            rc = rc + vf
        out_v[pl.ds(32, 16)] = rs
        out_v[pl.ds(48, 16)] = rc

    pltpu.sync_copy(out_v, part_r.at[wid])


def _fin_body(part_ref, pred0_ref, disp0_ref, pred1_ref, disp1_ref, o_ref):
    p = part_ref[...]
    shp = (_NUM_TILES, 128)
    rows = lax.broadcasted_iota(jnp.int32, shp, 0)
    cols = lax.broadcasted_iota(jnp.int32, shp, 1) // 16

    def msum(mask):
        return jnp.sum(jnp.where(mask, p, 0.0))

    total = jnp.float32(0.0)
    for g in range(4):
        gm = (rows // 8) == g
        num_g = msum(gm & (cols == 0))
        den_g = msum(gm & (cols == 1))
        total = total + num_g * jnp.float32(_N) / den_g
    loss_scoremap = total / jnp.float32(_B * 2 * _N)

    rsum = msum(cols == 2)
    rcnt = msum(cols == 3)
    loss_reproj = rsum / jnp.maximum(rcnt, 1.0)

    def pk(pred, disp):
        vf = jnp.where(pred > _TH, 1.0, 0.0)
        return jnp.sum(disp * vf) / jnp.maximum(jnp.sum(vf), 1.0)

    loss_peaky = (pk(pred0_ref[...], disp0_ref[...]) +
                  pk(pred1_ref[...], disp1_ref[...])) / 2.0

    o_ref[0] = (_PEAKY_W * loss_peaky + _REPROJ_W * loss_reproj +
                _SCOREMAP_W * loss_scoremap)


@jax.jit
def _detector_loss(scores_map0, scores_map1, scores_pred0, scores_pred1,
                   dispersity0, dispersity1, dist_l1, ids0_d, ids1_d,
                   scores0, scores1, kpts01, kpts10, sim01, sim10):
    sim01_f = sim01.reshape(-1)
    sim10_f = sim10.reshape(-1)
    smap0_f = scores_map0.reshape(-1)
    smap1_f = scores_map1.reshape(-1)
    dist2 = dist_l1.reshape(_B * _N, _N)
    aux = jnp.concatenate([kpts01[..., 0], kpts01[..., 1],
                           kpts10[..., 0], kpts10[..., 1],
                           scores0, scores1], axis=0)
    ids = jnp.pad(jnp.concatenate([ids0_d.astype(jnp.int32),
                                   ids1_d.astype(jnp.int32)], axis=0),
                  ((0, 0), (0, _N - _D)))

    mesh = plsc.VectorSubcoreMesh(core_axis_name="c", subcore_axis_name="s")
    sc_fn = pl.kernel(
        _sc_body,
        out_type=jax.ShapeDtypeStruct((_NUM_TILES, 128), jnp.float32),
        mesh=mesh,
        compiler_params=pltpu.CompilerParams(needs_layout_passes=False),
        scratch_types=[
            pltpu.VMEM((_N,), jnp.float32),
            pltpu.VMEM((_N,), jnp.float32),
            pltpu.VMEM((_N,), jnp.float32),
            pltpu.VMEM((_PTS_PER_TILE,), jnp.float32),
            pltpu.VMEM((_PTS_PER_TILE,), jnp.float32),
            pltpu.VMEM((4 * _PTS_PER_TILE,), jnp.int32),
            pltpu.VMEM((4 * _PTS_PER_TILE,), jnp.int32),
            pltpu.VMEM((4 * _PTS_PER_TILE,), jnp.float32),
            pltpu.VMEM((4 * _PTS_PER_TILE,), jnp.float32),
            pltpu.VMEM((4 * _PTS_PER_TILE,), jnp.float32),
            pltpu.VMEM((4 * _PTS_PER_TILE,), jnp.float32),
            pltpu.VMEM((_N,), jnp.int32),
            pltpu.VMEM((_N,), jnp.int32),
            pltpu.VMEM((_N,), jnp.float32),
            pltpu.VMEM((_N,), jnp.float32),
            pltpu.VMEM((_N, _N), jnp.float32),
            pltpu.VMEM((128,), jnp.float32),
            pltpu.SemaphoreType.DMA,
            pltpu.SemaphoreType.DMA,
            pltpu.SemaphoreType.DMA,
            pltpu.SemaphoreType.DMA,
            pltpu.SemaphoreType.DMA,
            pltpu.SemaphoreType.DMA,
            pltpu.SemaphoreType.DMA,
        ],
    )
    partials = sc_fn(sim01_f, sim10_f, smap0_f, smap1_f, aux, ids, dist2)

    loss = pl.pallas_call(
        _fin_body,
        out_shape=jax.ShapeDtypeStruct((1,), jnp.float32),
        out_specs=pl.BlockSpec(memory_space=pltpu.SMEM),
    )(partials, scores_pred0, dispersity0, scores_pred1, dispersity1)
    return loss[0]


def kernel(scores_map0, scores_map1, scores_pred0, scores_pred1, dispersity0,
           dispersity1, dist_l1, ids0_d, ids1_d, scores0, scores1, kpts01,
           kpts10, sim01, sim10):
    assert sim01.shape == (_B, _N, _H, _W)
    assert ids0_d.shape == (_B, _D)
    assert scores_pred0.shape == (_B, _M)
    return _detector_loss(scores_map0, scores_map1, scores_pred0,
                          scores_pred1, dispersity0, dispersity1, dist_l1,
                          ids0_d, ids1_d, scores0, scores1, kpts01, kpts10,
                          sim01, sim10)
